# Initial kernel scaffold; baseline (speedup 1.0000x reference)
import jax
import jax.numpy as jnp
from jax.experimental import pallas as pl


def kernel(feat0, feat1, indices0, dst0, indices1, dst1, W0, b0, W1, b1, r_vec, attn1_0, attn2_0, attn1_1, attn2_1, fc1_W, fc1_b, fc2_W, fc_W, fc_b):
    raise NotImplementedError("write your pallas kernel here")


# speedup vs baseline: 18.3134x; 18.3134x over previous
import jax
import jax.numpy as jnp
from jax.experimental import pallas as pl
from jax.experimental.pallas import tpu as pltpu

N0, N1 = 25000, 25000
N = N0 + N1
D = 128
H = 8
B = 2048
E = 65536
OUT = 64
AV = 128
EB = 2048
NBLK = E // EB
XB = 1000


def _xform_body(x_ref, wt_ref, b_ref, o_ref):
    o_ref[...] = (
        jnp.dot(x_ref[...], wt_ref[...], preferred_element_type=jnp.float32)
        + b_ref[...]
    )


def _transform(feat, W, b):
    n = feat.shape[0]
    return pl.pallas_call(
        _xform_body,
        grid=(n // XB,),
        in_specs=[
            pl.BlockSpec((XB, D), lambda i: (i, 0)),
            pl.BlockSpec((D, D), lambda i: (0, 0)),
            pl.BlockSpec((1, D), lambda i: (0, 0)),
        ],
        out_specs=pl.BlockSpec((XB, D), lambda i: (i, 0)),
        out_shape=jax.ShapeDtypeStruct((n, D), jnp.float32),
    )(feat, W.T, b.reshape(1, D))


CB = 512


def _metapath_body(tbl_ref, i0_ref, i1_ref, i2_ref, dst_ref,
                   r0_ref, r1_ref, a1t_ref, a2t_ref,
                   ret_ref, den_ref, g0_ref, g1_ref, g2_ref):
    step = pl.program_id(0)

    @pl.when(step == 0)
    def _init():
        ret_ref[...] = jnp.zeros_like(ret_ref)
        den_ref[...] = jnp.zeros_like(den_ref)

    def body(e, _):
        g0_ref[pl.ds(e, 1), :] = tbl_ref[pl.ds(i0_ref[0, 0, e], 1), :]
        g1_ref[pl.ds(e, 1), :] = tbl_ref[pl.ds(i1_ref[0, 0, e], 1), :]
        g2_ref[pl.ds(e, 1), :] = tbl_ref[pl.ds(i2_ref[0, 0, e], 1), :]
        return 0

    jax.lax.fori_loop(0, EB, body, 0)
    rows0 = g0_ref[...]
    rows1 = g1_ref[...]
    rows2 = g2_ref[...]

    hidden = (
        jnp.dot(rows0, r0_ref[...], preferred_element_type=jnp.float32)
        + jnp.dot(rows1, r1_ref[...], preferred_element_type=jnp.float32)
        + rows2 * (1.0 / 3.0)
    )

    a1 = jnp.dot(rows2, a1t_ref[...], preferred_element_type=jnp.float32)
    a2 = jnp.dot(hidden, a2t_ref[...], preferred_element_type=jnp.float32)
    a = a1 + a2
    a = jnp.where(a >= 0.0, a, 0.2 * a)
    ea = jnp.exp(a)

    dst = dst_ref[0]
    for c in range(B // CB):
        onehot = (
            jax.lax.broadcasted_iota(jnp.int32, (CB, EB), 0) + (c * CB) == dst
        ).astype(jnp.float32)
        den_ref[c * CB : (c + 1) * CB, :] += jnp.dot(
            onehot, ea, preferred_element_type=jnp.float32
        )
        for h in range(H):
            g = hidden * ea[:, h : h + 1]
            ret_ref[c * CB : (c + 1) * CB, h * D : (h + 1) * D] += jnp.dot(
                onehot, g, preferred_element_type=jnp.float32
            )

    @pl.when(step == NBLK - 1)
    def _fin():
        for h in range(H):
            x = ret_ref[:, h * D : (h + 1) * D] / jnp.maximum(
                den_ref[:, h : h + 1], 1e-12
            )
            ret_ref[:, h * D : (h + 1) * D] = jnp.where(
                x > 0.0, x, jnp.exp(x) - 1.0
            )


def _metapath(features, idx, dst, r0, r1, attn1, attn2):
    i0 = idx[:, 0].astype(jnp.int32).reshape(NBLK, 1, EB)
    i1 = idx[:, 1].astype(jnp.int32).reshape(NBLK, 1, EB)
    i2 = idx[:, 2].astype(jnp.int32).reshape(NBLK, 1, EB)
    dstr = dst.astype(jnp.int32).reshape(NBLK, 1, EB)
    a1t = jnp.zeros((D, D), jnp.float32).at[:, :H].set(attn1.T)
    a2t = jnp.zeros((D, D), jnp.float32).at[:, :H].set(attn2[0].T)
    idx_spec = pl.BlockSpec((1, 1, EB), lambda i: (i, 0, 0),
                            memory_space=pltpu.SMEM)
    dst_spec = pl.BlockSpec((1, 1, EB), lambda i: (i, 0, 0))
    full = lambda shape: pl.BlockSpec(shape, lambda i: tuple(0 for _ in shape))
    return pl.pallas_call(
        _metapath_body,
        grid=(NBLK,),
        in_specs=[
            full((N, D)),
            idx_spec, idx_spec, idx_spec, dst_spec,
            full((D, D)), full((D, D)), full((D, D)), full((D, D)),
        ],
        out_specs=full((B, H * D)),
        out_shape=jax.ShapeDtypeStruct((B, H * D), jnp.float32),
        scratch_shapes=[
            pltpu.VMEM((B, D), jnp.float32),
            pltpu.VMEM((EB, D), jnp.float32),
            pltpu.VMEM((EB, D), jnp.float32),
            pltpu.VMEM((EB, D), jnp.float32),
        ],
    )(features, i0, i1, i2, dstr, r0, r1, a1t, a2t)


def _tail_body(o0_ref, o1_ref, fc1wt_ref, fc1b_ref, fc2_ref,
               fcwt_ref, fcb_ref, logits_ref, h_ref, beta_ref):
    o0 = o0_ref[...]
    o1 = o1_ref[...]
    fc2 = fc2_ref[...]
    t0 = jnp.tanh(
        jnp.dot(o0, fc1wt_ref[...], preferred_element_type=jnp.float32)
        + fc1b_ref[...]
    )
    t1 = jnp.tanh(
        jnp.dot(o1, fc1wt_ref[...], preferred_element_type=jnp.float32)
        + fc1b_ref[...]
    )
    s0 = jnp.sum(t0 * fc2) / B
    s1 = jnp.sum(t1 * fc2) / B
    m = jnp.maximum(s0, s1)
    e0 = jnp.exp(s0 - m)
    e1 = jnp.exp(s1 - m)
    b0 = e0 / (e0 + e1)
    b1 = e1 / (e0 + e1)
    h = b0 * o0 + b1 * o1
    h_ref[...] = h
    logits_ref[...] = (
        jnp.dot(h, fcwt_ref[...], preferred_element_type=jnp.float32)
        + fcb_ref[...]
    )
    lane = jax.lax.broadcasted_iota(jnp.int32, (1, D), 1)
    beta_ref[...] = jnp.where(lane == 0, b0, jnp.where(lane == 1, b1, 0.0))


def kernel(feat0, feat1, indices0, dst0, indices1, dst1, W0, b0, W1, b1,
           r_vec, attn1_0, attn2_0, attn1_1, attn2_1, fc1_W, fc1_b, fc2_W,
           fc_W, fc_b):
    t0 = _transform(feat0, W0, b0)
    t1 = _transform(feat1, W1, b1)
    features = jnp.concatenate([t0, t1], axis=0)

    rn = r_vec / jnp.linalg.norm(r_vec, axis=2, keepdims=True)
    r = jnp.stack([rn, rn], axis=1)
    r = r.at[:, 1, :, 1].multiply(-1.0)
    r = r.reshape(4, D // 2, 2)

    def _cmul(x, y):
        re = x[..., 0] * y[..., 0] - x[..., 1] * y[..., 1]
        im = x[..., 0] * y[..., 1] + x[..., 1] * y[..., 0]
        return jnp.stack([re, im], axis=-1)

    def _rotmat(f):
        fre = f[:, 0]
        fim = f[:, 1]
        base = jnp.arange(D // 2) * 2
        m = jnp.zeros((D, D), jnp.float32)
        m = m.at[base, base].set(fre)
        m = m.at[base + 1, base].set(-fim)
        m = m.at[base, base + 1].set(fim)
        m = m.at[base + 1, base + 1].set(fre)
        return m

    outs = []
    for idx, dst, etypes, attn1, attn2 in (
        (indices0, dst0, (0, 1), attn1_0, attn2_0),
        (indices1, dst1, (2, 3), attn1_1, attn2_1),
    ):
        fin1 = r[etypes[1]]
        fin0 = _cmul(fin1, r[etypes[0]])
        r0m = _rotmat(fin0) * (1.0 / 3.0)
        r1m = _rotmat(fin1) * (1.0 / 3.0)
        outs.append(_metapath(features, idx, dst, r0m, r1m, attn1, attn2))

    out0, out1 = outs
    logits, h, beta_row = pl.pallas_call(
        _tail_body,
        in_specs=[pl.BlockSpec(s, lambda: tuple(0 for _ in s)) for s in (
            (B, H * D), (B, H * D), (H * D, AV), (1, AV), (1, AV),
            (H * D, OUT), (1, OUT),
        )],
        out_specs=[
            pl.BlockSpec((B, OUT), lambda: (0, 0)),
            pl.BlockSpec((B, H * D), lambda: (0, 0)),
            pl.BlockSpec((1, D), lambda: (0, 0)),
        ],
        out_shape=[
            jax.ShapeDtypeStruct((B, OUT), jnp.float32),
            jax.ShapeDtypeStruct((B, H * D), jnp.float32),
            jax.ShapeDtypeStruct((1, D), jnp.float32),
        ],
    )(out0, out1, fc1_W.T, fc1_b.reshape(1, AV), fc2_W, fc_W.T,
      fc_b.reshape(1, OUT))
    beta = beta_row[0, :2]
    return (logits, h, beta)


# speedup vs baseline: 24.8125x; 1.3549x over previous
import jax
import jax.numpy as jnp
from jax.experimental import pallas as pl
from jax.experimental.pallas import tpu as pltpu

N0, N1 = 25000, 25000
N = N0 + N1
D = 128
H = 8
B = 2048
E = 65536
OUT = 64
AV = 128
EB = 2048
NBLK = E // EB
XB = 1000


def _xform_body(x_ref, wt_ref, b_ref, o_ref):
    o_ref[...] = (
        jnp.dot(x_ref[...], wt_ref[...], preferred_element_type=jnp.float32)
        + b_ref[...]
    )


def _transform(feat, W, b):
    n = feat.shape[0]
    return pl.pallas_call(
        _xform_body,
        grid=(n // XB,),
        in_specs=[
            pl.BlockSpec((XB, D), lambda i: (i, 0)),
            pl.BlockSpec((D, D), lambda i: (0, 0)),
            pl.BlockSpec((1, D), lambda i: (0, 0)),
        ],
        out_specs=pl.BlockSpec((XB, D), lambda i: (i, 0)),
        out_shape=jax.ShapeDtypeStruct((n, D), jnp.float32),
    )(feat, W.T, b.reshape(1, D))


CB = 512


def _metapath_body(tbl_ref, i0_ref, i1_ref, i2_ref, dst_ref,
                   r0_ref, r1_ref, a1t_ref, a2t_ref,
                   ret_ref, den_ref, g0_ref, g1_ref, g2_ref):
    step = pl.program_id(0)

    @pl.when(step == 0)
    def _init():
        ret_ref[...] = jnp.zeros_like(ret_ref)
        den_ref[...] = jnp.zeros_like(den_ref)

    def body(e, _):
        g0_ref[pl.ds(e, 1), :] = tbl_ref[pl.ds(i0_ref[0, 0, e], 1), :]
        g1_ref[pl.ds(e, 1), :] = tbl_ref[pl.ds(i1_ref[0, 0, e], 1), :]
        g2_ref[pl.ds(e, 1), :] = tbl_ref[pl.ds(i2_ref[0, 0, e], 1), :]
        return 0

    jax.lax.fori_loop(0, EB, body, 0, unroll=8)
    rows0 = g0_ref[...]
    rows1 = g1_ref[...]
    rows2 = g2_ref[...]

    hidden = (
        jnp.dot(rows0, r0_ref[...], preferred_element_type=jnp.float32)
        + jnp.dot(rows1, r1_ref[...], preferred_element_type=jnp.float32)
        + rows2 * (1.0 / 3.0)
    )

    a1 = jnp.dot(rows2, a1t_ref[...], preferred_element_type=jnp.float32)
    a2 = jnp.dot(hidden, a2t_ref[...], preferred_element_type=jnp.float32)
    a = a1 + a2
    a = jnp.where(a >= 0.0, a, 0.2 * a)
    ea = jnp.exp(a)

    dst = dst_ref[0]
    for c in range(B // CB):
        onehot = (
            jax.lax.broadcasted_iota(jnp.int32, (CB, EB), 0) + (c * CB) == dst
        ).astype(jnp.float32)
        den_ref[c * CB : (c + 1) * CB, :] += jnp.dot(
            onehot, ea, preferred_element_type=jnp.float32
        )
        for h in range(H):
            g = hidden * ea[:, h : h + 1]
            ret_ref[c * CB : (c + 1) * CB, h * D : (h + 1) * D] += jnp.dot(
                onehot, g, preferred_element_type=jnp.float32
            )

    @pl.when(step == NBLK - 1)
    def _fin():
        for h in range(H):
            x = ret_ref[:, h * D : (h + 1) * D] / jnp.maximum(
                den_ref[:, h : h + 1], 1e-12
            )
            ret_ref[:, h * D : (h + 1) * D] = jnp.where(
                x > 0.0, x, jnp.exp(x) - 1.0
            )


def _metapath(features, idx, dst, r0, r1, attn1, attn2):
    i0 = idx[:, 0].astype(jnp.int32).reshape(NBLK, 1, EB)
    i1 = idx[:, 1].astype(jnp.int32).reshape(NBLK, 1, EB)
    i2 = idx[:, 2].astype(jnp.int32).reshape(NBLK, 1, EB)
    dstr = dst.astype(jnp.int32).reshape(NBLK, 1, EB)
    a1t = jnp.zeros((D, D), jnp.float32).at[:, :H].set(attn1.T)
    a2t = jnp.zeros((D, D), jnp.float32).at[:, :H].set(attn2[0].T)
    idx_spec = pl.BlockSpec((1, 1, EB), lambda i: (i, 0, 0),
                            memory_space=pltpu.SMEM)
    dst_spec = pl.BlockSpec((1, 1, EB), lambda i: (i, 0, 0))
    full = lambda shape: pl.BlockSpec(shape, lambda i: tuple(0 for _ in shape))
    return pl.pallas_call(
        _metapath_body,
        grid=(NBLK,),
        in_specs=[
            full((N, D)),
            idx_spec, idx_spec, idx_spec, dst_spec,
            full((D, D)), full((D, D)), full((D, D)), full((D, D)),
        ],
        out_specs=full((B, H * D)),
        out_shape=jax.ShapeDtypeStruct((B, H * D), jnp.float32),
        scratch_shapes=[
            pltpu.VMEM((B, D), jnp.float32),
            pltpu.VMEM((EB, D), jnp.float32),
            pltpu.VMEM((EB, D), jnp.float32),
            pltpu.VMEM((EB, D), jnp.float32),
        ],
    )(features, i0, i1, i2, dstr, r0, r1, a1t, a2t)


def _tail_body(o0_ref, o1_ref, fc1wt_ref, fc1b_ref, fc2_ref,
               fcwt_ref, fcb_ref, logits_ref, h_ref, beta_ref):
    o0 = o0_ref[...]
    o1 = o1_ref[...]
    fc2 = fc2_ref[...]
    t0 = jnp.tanh(
        jnp.dot(o0, fc1wt_ref[...], preferred_element_type=jnp.float32)
        + fc1b_ref[...]
    )
    t1 = jnp.tanh(
        jnp.dot(o1, fc1wt_ref[...], preferred_element_type=jnp.float32)
        + fc1b_ref[...]
    )
    s0 = jnp.sum(t0 * fc2) / B
    s1 = jnp.sum(t1 * fc2) / B
    m = jnp.maximum(s0, s1)
    e0 = jnp.exp(s0 - m)
    e1 = jnp.exp(s1 - m)
    b0 = e0 / (e0 + e1)
    b1 = e1 / (e0 + e1)
    h = b0 * o0 + b1 * o1
    h_ref[...] = h
    logits_ref[...] = (
        jnp.dot(h, fcwt_ref[...], preferred_element_type=jnp.float32)
        + fcb_ref[...]
    )
    lane = jax.lax.broadcasted_iota(jnp.int32, (1, D), 1)
    beta_ref[...] = jnp.where(lane == 0, b0, jnp.where(lane == 1, b1, 0.0))


def kernel(feat0, feat1, indices0, dst0, indices1, dst1, W0, b0, W1, b1,
           r_vec, attn1_0, attn2_0, attn1_1, attn2_1, fc1_W, fc1_b, fc2_W,
           fc_W, fc_b):
    t0 = _transform(feat0, W0, b0)
    t1 = _transform(feat1, W1, b1)
    features = jnp.concatenate([t0, t1], axis=0)

    rn = r_vec / jnp.linalg.norm(r_vec, axis=2, keepdims=True)
    r = jnp.stack([rn, rn], axis=1)
    r = r.at[:, 1, :, 1].multiply(-1.0)
    r = r.reshape(4, D // 2, 2)

    def _cmul(x, y):
        re = x[..., 0] * y[..., 0] - x[..., 1] * y[..., 1]
        im = x[..., 0] * y[..., 1] + x[..., 1] * y[..., 0]
        return jnp.stack([re, im], axis=-1)

    def _rotmat(f):
        fre = f[:, 0]
        fim = f[:, 1]
        base = jnp.arange(D // 2) * 2
        m = jnp.zeros((D, D), jnp.float32)
        m = m.at[base, base].set(fre)
        m = m.at[base + 1, base].set(-fim)
        m = m.at[base, base + 1].set(fim)
        m = m.at[base + 1, base + 1].set(fre)
        return m

    outs = []
    for idx, dst, etypes, attn1, attn2 in (
        (indices0, dst0, (0, 1), attn1_0, attn2_0),
        (indices1, dst1, (2, 3), attn1_1, attn2_1),
    ):
        fin1 = r[etypes[1]]
        fin0 = _cmul(fin1, r[etypes[0]])
        r0m = _rotmat(fin0) * (1.0 / 3.0)
        r1m = _rotmat(fin1) * (1.0 / 3.0)
        outs.append(_metapath(features, idx, dst, r0m, r1m, attn1, attn2))

    out0, out1 = outs
    logits, h, beta_row = pl.pallas_call(
        _tail_body,
        in_specs=[pl.BlockSpec(s, lambda: tuple(0 for _ in s)) for s in (
            (B, H * D), (B, H * D), (H * D, AV), (1, AV), (1, AV),
            (H * D, OUT), (1, OUT),
        )],
        out_specs=[
            pl.BlockSpec((B, OUT), lambda: (0, 0)),
            pl.BlockSpec((B, H * D), lambda: (0, 0)),
            pl.BlockSpec((1, D), lambda: (0, 0)),
        ],
        out_shape=[
            jax.ShapeDtypeStruct((B, OUT), jnp.float32),
            jax.ShapeDtypeStruct((B, H * D), jnp.float32),
            jax.ShapeDtypeStruct((1, D), jnp.float32),
        ],
    )(out0, out1, fc1_W.T, fc1_b.reshape(1, AV), fc2_W, fc_W.T,
      fc_b.reshape(1, OUT))
    beta = beta_row[0, :2]
    return (logits, h, beta)
